# baseline (speedup 1.0000x reference)
import functools

import jax
import jax.numpy as jnp
from jax import lax
from jax.experimental import pallas as pl
from jax.experimental.pallas import tpu as pltpu
from jax.experimental.pallas import tpu_sc as plsc

_B_OUT = 4096
_LANES = 16
_SUB = 8
_LANE_T = 128


@functools.lru_cache(maxsize=None)
def _build(t_rows: int, dim: int):
    mesh = plsc.VectorSubcoreMesh(core_axis_name="c", subcore_axis_name="s")
    nc, ns = mesh.num_cores, mesh.num_subcores
    nw = nc * ns
    n_bblk = _B_OUT // _LANE_T
    n_dblk = dim // _SUB
    n_units = t_rows * n_dblk
    unit_words = n_bblk * _SUB * _LANE_T
    assert n_units % nw == 0
    units_per_w = n_units // nw
    n_cycles = (units_per_w - 2) // 6
    assert n_cycles * 6 + 2 == units_per_w
    vals_per_w = units_per_w * _SUB

    @functools.partial(
        pl.kernel,
        mesh=mesh,
        out_type=jax.ShapeDtypeStruct((n_units * unit_words,), jnp.float32),
        scratch_types=[
            pltpu.VMEM((vals_per_w,), jnp.float32),
            pltpu.VMEM((dim,), jnp.float32),
            pltpu.VMEM((2 * unit_words,), jnp.float32),
            pltpu.VMEM((unit_words,), jnp.float32),
            pltpu.SemaphoreType.DMA,
            pltpu.SemaphoreType.DMA,
        ],
    )
    def k(temb_hbm, bias_hbm, out_hbm, tab_v, bias_v, buf_a, buf_b,
          sem_a, sem_b):
        wid = lax.axis_index("s") * nc + lax.axis_index("c")
        u0 = wid * units_per_w
        v0 = u0 * _SUB
        pltpu.sync_copy(temb_hbm.at[pl.ds(v0, vals_per_w)], tab_v)
        pltpu.sync_copy(bias_hbm, bias_v)

        def splat_pair(pair_local):
            vec = tab_v[pl.ds(_LANES * pair_local, _LANES)]
            bvec = bias_v[pl.ds((v0 + _LANES * pair_local) % dim, _LANES)]
            sv = vec + bvec
            return [jnp.full((_LANES,), sv[i], jnp.float32) for i in range(_LANES)]

        def build_unit(vals, buf, slot):
            def rep_body(rep, carry):
                off = slot * unit_words + rep * _SUB * _LANE_T
                for r in range(_SUB):
                    for j in range(_LANE_T // _LANES):
                        buf[pl.ds(off + r * _LANE_T + j * _LANES, _LANES)] = vals[r]
                return carry

            lax.fori_loop(0, n_bblk, rep_body, 0)

        def fire(buf, n_u, u, sem):
            pltpu.async_copy(
                buf,
                out_hbm.at[pl.ds((u0 + u) * unit_words, n_u * unit_words)],
                sem,
            )

        def wait(buf, n_u, u, sem):
            pltpu.make_async_copy(
                buf,
                out_hbm.at[pl.ds((u0 + u) * unit_words, n_u * unit_words)],
                sem,
            ).wait()

        def cycle(c, carry):
            base = 6 * c
            sv0 = splat_pair(3 * c)
            sv1 = splat_pair(3 * c + 1)
            sv2 = splat_pair(3 * c + 2)

            @pl.when(c > 0)
            def _wa():
                wait(buf_a, 2, base - 3, sem_a)

            build_unit(sv0[:_SUB], buf_a, 0)
            build_unit(sv0[_SUB:], buf_a, 1)
            fire(buf_a, 2, base, sem_a)

            @pl.when(c > 0)
            def _wb():
                wait(buf_b, 1, base - 1, sem_b)

            build_unit(sv1[:_SUB], buf_b, 0)
            fire(buf_b, 1, base + 2, sem_b)

            wait(buf_a, 2, base, sem_a)
            build_unit(sv1[_SUB:], buf_a, 0)
            build_unit(sv2[:_SUB], buf_a, 1)
            fire(buf_a, 2, base + 3, sem_a)

            wait(buf_b, 1, base + 2, sem_b)
            build_unit(sv2[_SUB:], buf_b, 0)
            fire(buf_b, 1, base + 5, sem_b)
            return carry

        lax.fori_loop(0, n_cycles, cycle, 0)

        tail = 6 * n_cycles
        svt = splat_pair(3 * n_cycles)
        wait(buf_a, 2, tail - 3, sem_a)
        build_unit(svt[:_SUB], buf_a, 0)
        build_unit(svt[_SUB:], buf_a, 1)
        fire(buf_a, 2, tail, sem_a)
        wait(buf_b, 1, tail - 1, sem_b)
        wait(buf_a, 2, tail, sem_a)

    return k


def kernel(B, T, temb_weight, type_bias):
    t_rows, dim = temb_weight.shape
    temb_flat = temb_weight.reshape(t_rows * dim)
    bias_flat = type_bias.reshape(dim)
    out = _build(t_rows, dim)(temb_flat, bias_flat)
    n_dblk = dim // _SUB
    n_bblk = _B_OUT // _LANE_T
    out = out.reshape(t_rows, n_dblk, n_bblk, _SUB, _LANE_T)
    out = out.transpose(2, 4, 0, 1, 3).reshape(_B_OUT, t_rows, dim)
    return out[:, :, None, :]


# speedup vs baseline: 1.3809x; 1.3809x over previous
import functools

import jax
import jax.numpy as jnp
from jax import lax
from jax.experimental import pallas as pl
from jax.experimental.pallas import tpu as pltpu
from jax.experimental.pallas import tpu_sc as plsc

_B_OUT = 4096
_LANES = 16
_SUB = 8
_LANE_T = 128


@functools.lru_cache(maxsize=None)
def _build(t_rows: int, dim: int):
    mesh = plsc.VectorSubcoreMesh(core_axis_name="c", subcore_axis_name="s")
    nc, ns = mesh.num_cores, mesh.num_subcores
    nw = nc * ns
    n_bblk = _B_OUT // _LANE_T
    n_dblk = dim // _SUB
    n_units = t_rows * n_dblk
    assert n_units % nw == 0
    units_per_w = n_units // nw
    assert units_per_w % 2 == 0
    vals_per_w = units_per_w * _SUB

    @functools.partial(
        pl.kernel,
        mesh=mesh,
        out_type=jax.ShapeDtypeStruct(
            (t_rows, n_dblk, n_bblk, _SUB, _LANE_T), jnp.float32
        ),
        scratch_types=[
            pltpu.VMEM((vals_per_w,), jnp.float32),
            pltpu.VMEM((dim,), jnp.float32),
            pltpu.VMEM((n_bblk, _SUB, _LANE_T), jnp.float32),
            pltpu.VMEM((n_bblk, _SUB, _LANE_T), jnp.float32),
            pltpu.SemaphoreType.DMA,
            pltpu.SemaphoreType.DMA,
        ],
    )
    def k(temb_hbm, bias_hbm, out_hbm, tab_v, bias_v, buf_a, buf_b,
          sem_a, sem_b):
        wid = lax.axis_index("s") * nc + lax.axis_index("c")
        u0 = wid * units_per_w
        v0 = u0 * _SUB
        pltpu.sync_copy(temb_hbm.at[pl.ds(v0, vals_per_w)], tab_v)
        pltpu.sync_copy(bias_hbm, bias_v)

        def build_unit(vals, buf):
            def rep_body(rep, carry):
                for r in range(_SUB):
                    for j in range(_LANE_T // _LANES):
                        buf[rep, r, pl.ds(j * _LANES, _LANES)] = vals[r]
                return carry

            lax.fori_loop(0, n_bblk, rep_body, 0)

        def pair(p, carry):
            vec = tab_v[pl.ds(_LANES * p, _LANES)]
            bvec = bias_v[pl.ds((v0 + _LANES * p) % dim, _LANES)]
            sv = vec + bvec
            for b, buf, sem in ((0, buf_a, sem_a), (1, buf_b, sem_b)):
                u = u0 + 2 * p + b

                @pl.when(p > 0)
                def _wait():
                    up = u - 2
                    pltpu.make_async_copy(
                        buf, out_hbm.at[up // n_dblk, up % n_dblk], sem
                    ).wait()

                vals = [
                    jnp.full((_LANES,), sv[_SUB * b + r], jnp.float32)
                    for r in range(_SUB)
                ]
                build_unit(vals, buf)
                pltpu.async_copy(buf, out_hbm.at[u // n_dblk, u % n_dblk], sem)
            return carry

        lax.fori_loop(0, units_per_w // 2, pair, 0)
        ua = u0 + units_per_w - 2
        ub = u0 + units_per_w - 1
        pltpu.make_async_copy(
            buf_a, out_hbm.at[ua // n_dblk, ua % n_dblk], sem_a
        ).wait()
        pltpu.make_async_copy(
            buf_b, out_hbm.at[ub // n_dblk, ub % n_dblk], sem_b
        ).wait()

    return k


def kernel(B, T, temb_weight, type_bias):
    t_rows, dim = temb_weight.shape
    temb_flat = temb_weight.reshape(t_rows * dim)
    bias_flat = type_bias.reshape(dim)
    out = _build(t_rows, dim)(temb_flat, bias_flat)
    out = out.transpose(2, 4, 0, 1, 3).reshape(_B_OUT, t_rows, dim)
    return out[:, :, None, :]
